# baseline (speedup 1.0000x reference)
import jax
import jax.numpy as jnp
from jax.experimental import pallas as pl
from jax.experimental.pallas import tpu as pltpu

_BT = 2048


def _router_body(x_ref, wt_ref, logits_ref, w_ref, e_ref):
    x = x_ref[...]
    logits = jnp.dot(x, wt_ref[...], preferred_element_type=jnp.float32)
    logits_ref[...] = logits
    ncols = logits.shape[1]
    idx = jax.lax.broadcasted_iota(jnp.int32, logits.shape, 1)
    m1 = jnp.max(logits, axis=1, keepdims=True)
    a1 = jnp.min(jnp.where(logits == m1, idx, ncols), axis=1, keepdims=True)
    masked = jnp.where(idx == a1, -jnp.inf, logits)
    m2 = jnp.max(masked, axis=1, keepdims=True)
    a2 = jnp.min(jnp.where(masked == m2, idx, ncols), axis=1, keepdims=True)
    w1 = 1.0 / (1.0 + jnp.exp(m2 - m1))
    w2 = 1.0 - w1
    w_ref[...] = jnp.concatenate([w1, w2], axis=1)
    e_ref[...] = jnp.concatenate([a1, a2], axis=1)


def kernel(hidden_states, W_router):
    b, s, h = hidden_states.shape
    n_exp = W_router.shape[0]
    n_tok = b * s
    x = hidden_states.reshape(n_tok, h)
    wt = W_router.T

    grid = (n_tok // _BT,)
    logits, weights, experts = pl.pallas_call(
        _router_body,
        grid=grid,
        in_specs=[
            pl.BlockSpec((_BT, h), lambda i: (i, 0)),
            pl.BlockSpec((h, n_exp), lambda i: (0, 0)),
        ],
        out_specs=[
            pl.BlockSpec((_BT, n_exp), lambda i: (i, 0)),
            pl.BlockSpec((_BT, 2), lambda i: (i, 0)),
            pl.BlockSpec((_BT, 2), lambda i: (i, 0)),
        ],
        out_shape=[
            jax.ShapeDtypeStruct((n_tok, n_exp), jnp.float32),
            jax.ShapeDtypeStruct((n_tok, 2), jnp.float32),
            jax.ShapeDtypeStruct((n_tok, 2), jnp.int32),
        ],
    )(x, wt)

    return (
        weights.reshape(b, s, 2),
        experts.reshape(b, s, 2),
        logits.reshape(b, s, n_exp),
    )


# speedup vs baseline: 1.0912x; 1.0912x over previous
import jax
import jax.numpy as jnp
from jax.experimental import pallas as pl
from jax.experimental.pallas import tpu as pltpu

_BT = 2048


def _router_body(x_ref, w_router_ref, logits_ref, w_ref, e_ref):
    logits_t = jax.lax.dot_general(
        w_router_ref[...], x_ref[...],
        dimension_numbers=(((1,), (1,)), ((), ())),
        preferred_element_type=jnp.float32,
    )
    logits_ref[...] = logits_t.T
    nexp = logits_t.shape[0]
    idx = jax.lax.broadcasted_iota(jnp.int32, logits_t.shape, 0)
    m1 = jnp.max(logits_t, axis=0, keepdims=True)
    a1 = jnp.min(jnp.where(logits_t == m1, idx, nexp), axis=0, keepdims=True)
    masked = jnp.where(idx == a1, -jnp.inf, logits_t)
    m2 = jnp.max(masked, axis=0, keepdims=True)
    a2 = jnp.min(jnp.where(masked == m2, idx, nexp), axis=0, keepdims=True)
    w1 = 1.0 / (1.0 + jnp.exp(m2 - m1))
    w2 = 1.0 - w1
    w_ref[...] = jnp.concatenate([w1, w2], axis=0).T
    e_ref[...] = jnp.concatenate([a1, a2], axis=0).T


def kernel(hidden_states, W_router):
    b, s, h = hidden_states.shape
    n_exp = W_router.shape[0]
    n_tok = b * s
    x = hidden_states.reshape(n_tok, h)

    grid = (n_tok // _BT,)
    logits, weights, experts = pl.pallas_call(
        _router_body,
        grid=grid,
        in_specs=[
            pl.BlockSpec((_BT, h), lambda i: (i, 0)),
            pl.BlockSpec((n_exp, h), lambda i: (0, 0)),
        ],
        out_specs=[
            pl.BlockSpec((_BT, n_exp), lambda i: (i, 0)),
            pl.BlockSpec((_BT, 2), lambda i: (i, 0)),
            pl.BlockSpec((_BT, 2), lambda i: (i, 0)),
        ],
        out_shape=[
            jax.ShapeDtypeStruct((n_tok, n_exp), jnp.float32),
            jax.ShapeDtypeStruct((n_tok, 2), jnp.float32),
            jax.ShapeDtypeStruct((n_tok, 2), jnp.int32),
        ],
    )(x, W_router)

    return (
        weights.reshape(b, s, 2),
        experts.reshape(b, s, 2),
        logits.reshape(b, s, n_exp),
    )


# speedup vs baseline: 1.0931x; 1.0017x over previous
import jax
import jax.numpy as jnp
from jax.experimental import pallas as pl
from jax.experimental.pallas import tpu as pltpu

_S = 4
_BT = 1024


def _router_body(*refs):
    x_refs = refs[:_S]
    w_router_ref = refs[_S]
    logits_ref, w_ref, e_ref = refs[_S + 1:]
    w_router = w_router_ref[...]
    nexp = w_router.shape[0]
    for s in range(_S):
        logits_t = jax.lax.dot_general(
            w_router, x_refs[s][0],
            dimension_numbers=(((1,), (1,)), ((), ())),
            preferred_element_type=jnp.float32,
        )
        logits_ref[s] = logits_t.T
        idx = jax.lax.broadcasted_iota(jnp.int32, logits_t.shape, 0)
        m1 = jnp.max(logits_t, axis=0, keepdims=True)
        a1 = jnp.min(jnp.where(logits_t == m1, idx, nexp), axis=0, keepdims=True)
        masked = jnp.where(idx == a1, -jnp.inf, logits_t)
        m2 = jnp.max(masked, axis=0, keepdims=True)
        a2 = jnp.min(jnp.where(masked == m2, idx, nexp), axis=0, keepdims=True)
        w1 = 1.0 / (1.0 + jnp.exp(m2 - m1))
        w2 = 1.0 - w1
        w_ref[s] = jnp.concatenate([w1, w2], axis=0).T
        e_ref[s] = jnp.concatenate([a1, a2], axis=0).T


def kernel(hidden_states, W_router):
    b, s, h = hidden_states.shape
    n_exp = W_router.shape[0]
    n_tok = b * s
    rows = n_tok // _S
    x = hidden_states.reshape(_S, rows, h)

    grid = (rows // _BT,)
    in_specs = [
        pl.BlockSpec((1, _BT, h), (lambda i, sl=sl: (sl, i, 0)))
        for sl in range(_S)
    ] + [pl.BlockSpec((n_exp, h), lambda i: (0, 0))]
    logits, weights, experts = pl.pallas_call(
        _router_body,
        grid=grid,
        in_specs=in_specs,
        out_specs=[
            pl.BlockSpec((_S, _BT, n_exp), lambda i: (0, i, 0)),
            pl.BlockSpec((_S, _BT, 2), lambda i: (0, i, 0)),
            pl.BlockSpec((_S, _BT, 2), lambda i: (0, i, 0)),
        ],
        out_shape=[
            jax.ShapeDtypeStruct((_S, rows, n_exp), jnp.float32),
            jax.ShapeDtypeStruct((_S, rows, 2), jnp.float32),
            jax.ShapeDtypeStruct((_S, rows, 2), jnp.int32),
        ],
    )(*([x] * _S), W_router)

    return (
        weights.reshape(b, s, 2),
        experts.reshape(b, s, 2),
        logits.reshape(b, s, n_exp),
    )
